# baseline (speedup 1.0000x reference)
import functools

import jax
import jax.numpy as jnp
from jax import lax
from jax.experimental import pallas as pl
from jax.experimental.pallas import tpu as pltpu
from jax.experimental.pallas import tpu_sc as plsc

_NC = 2
_NS = 16
_LANE = 128


def _edge_block(e_per_s: int) -> int:
    for b in (128, 120, 112, 104, 96, 88, 80, 72, 64, 56, 48, 40, 32, 24, 16, 8):
        if e_per_s % b == 0:
            return b
    return 0


@functools.lru_cache(maxsize=None)
def _seg_accum(C: int, N: int, E: int):
    assert C % _NC == 0 and E % _NS == 0
    cpc = C // _NC
    e_per_s = E // _NS
    B = _edge_block(e_per_s)
    assert B > 0
    nblk = e_per_s // B
    ph0 = (nblk // 2 + 7) // 8 * 8
    if ph0 >= nblk:
        ph0 = nblk - 8
    phases = ((0, ph0), (ph0, nblk - ph0))
    ph_max = max(ph0, nblk - ph0)

    mesh = plsc.VectorSubcoreMesh(
        core_axis_name="c", subcore_axis_name="s", num_cores=_NC, num_subcores=_NS
    )

    n_hi = ((N // _NS) + 7) // 8 * 8
    n_lo = N - (_NS - 1) * n_hi
    assert n_lo > 0 and n_lo % 8 == 0

    def _row_copy(src_ref, dst_ref, src_base, dst_base, sub):
        @pl.when(sub < _NS - 1)
        def _():
            s = pl.multiple_of(src_base + sub * n_hi, 8)
            d = pl.multiple_of(dst_base + sub * n_hi, 8)
            pltpu.sync_copy(src_ref.at[pl.ds(s, n_hi)], dst_ref.at[pl.ds(d, n_hi)])

        @pl.when(sub == _NS - 1)
        def _():
            s = pl.multiple_of(src_base + (_NS - 1) * n_hi, 8)
            d = pl.multiple_of(dst_base + (_NS - 1) * n_hi, 8)
            pltpu.sync_copy(src_ref.at[pl.ds(s, n_lo)], dst_ref.at[pl.ds(d, n_lo)])

    @functools.partial(
        pl.kernel,
        out_type=jax.ShapeDtypeStruct((C * N, _LANE), jnp.float32),
        mesh=mesh,
        scratch_types=[
            pltpu.VMEM((ph_max, B), jnp.int32),
            pltpu.VMEM((nblk, B), jnp.int32),
            pltpu.VMEM((B, _LANE), jnp.float32),
            pltpu.VMEM((B, _LANE), jnp.float32),
            pltpu.VMEM_SHARED((N, _LANE), jnp.float32),
            pltpu.SemaphoreType.DMA,
            pltpu.SemaphoreType.DMA,
            pltpu.SemaphoreType.DMA,
            pltpu.SemaphoreType.DMA,
        ],
    )
    def seg_kernel(x_hbm, srco_hbm, dst_hbm, out_hbm, src_v, dst_v,
                   rows0, rows1, acc_sh, gs0, gs1, ss0, ss1):
        core = lax.axis_index("c")
        sub = lax.axis_index("s")
        pltpu.sync_copy(dst_hbm.at[sub], dst_v)

        def pipe_phase(base, n, c):
            def g_start(i, buf, sem):
                pltpu.async_copy(x_hbm.at[src_v.at[i]], buf, sem)

            def g_wait(i, buf, sem):
                pltpu.make_async_copy(x_hbm.at[src_v.at[i]], buf, sem).wait()

            def s_start(i, buf, sem):
                pltpu.async_copy(buf, acc_sh.at[dst_v.at[base + i]], sem, add=True)

            def s_wait(i, buf, sem):
                pltpu.make_async_copy(buf, acc_sh.at[dst_v.at[base + i]], sem).wait()

            g_start(0, rows0, gs0)
            g_start(1, rows1, gs1)
            n_pairs = (n - 2) // 2

            @pl.loop(0, n_pairs)
            def _(t):
                i = t * 2
                g_wait(i, rows0, gs0)
                s_start(i, rows0, ss0)
                g_wait(i + 1, rows1, gs1)
                s_start(i + 1, rows1, ss1)
                s_wait(i, rows0, ss0)
                g_start(i + 2, rows0, gs0)
                s_wait(i + 1, rows1, ss1)
                g_start(i + 3, rows1, gs1)

            r = 2 * n_pairs
            g_wait(r, rows0, gs0)
            s_start(r, rows0, ss0)
            g_wait(r + 1, rows1, gs1)
            s_start(r + 1, rows1, ss1)
            s_wait(r, rows0, ss0)
            if n - r == 3:
                g_start(r + 2, rows0, gs0)
            s_wait(r + 1, rows1, ss1)
            if n - r == 3:
                g_wait(r + 2, rows0, gs0)
                s_start(r + 2, rows0, ss0)
                s_wait(r + 2, rows0, ss0)

        for j in range(cpc):
            c = core * cpc + j
            _row_copy(x_hbm, acc_sh, c * N, 0, sub)
            plsc.subcore_barrier()

            for base, n_ph in phases:
                pltpu.sync_copy(
                    srco_hbm.at[c * _NS + sub].at[pl.ds(base, n_ph)],
                    src_v.at[pl.ds(0, n_ph)],
                )
                pipe_phase(base, n_ph, c)

            plsc.subcore_barrier()
            _row_copy(acc_sh, out_hbm, 0, c * N, sub)

    return seg_kernel


@functools.lru_cache(maxsize=None)
def _gin_linear(C_in: int, C_out: int, N: int, BN: int):
    D_out = C_out * _LANE
    grid = (N // BN,)

    def body(s_ref, w_ref, b_ref, o_ref):
        acc = jnp.dot(s_ref[0], w_ref[0], preferred_element_type=jnp.float32)
        for c in range(1, C_in):
            acc += jnp.dot(s_ref[c], w_ref[c], preferred_element_type=jnp.float32)
        acc = acc + b_ref[...]
        h = jnp.where(acc >= 0, acc, 0.01 * acc)
        for j in range(C_out):
            o_ref[j] = h[:, j * _LANE:(j + 1) * _LANE]

    return pl.pallas_call(
        body,
        grid=grid,
        in_specs=[
            pl.BlockSpec((C_in, BN, _LANE), lambda i: (0, i, 0)),
            pl.BlockSpec((C_in, _LANE, D_out), lambda i: (0, 0, 0)),
            pl.BlockSpec((1, D_out), lambda i: (0, 0)),
        ],
        out_specs=pl.BlockSpec((C_out, BN, _LANE), lambda i: (0, i, 0)),
        out_shape=jax.ShapeDtypeStruct((C_out, N, _LANE), jnp.float32),
    )


@functools.lru_cache(maxsize=None)
def _gin_final(C_in: int, N: int, BN: int, n_classes: int):
    D_h = 512
    grid = (N // BN,)

    def body(s_ref, w2_ref, b2_ref, w3_ref, b3_ref, o_ref, acc_ref):
        i = pl.program_id(0)
        z = jnp.dot(s_ref[0], w2_ref[0], preferred_element_type=jnp.float32)
        for c in range(1, C_in):
            z += jnp.dot(s_ref[c], w2_ref[c], preferred_element_type=jnp.float32)
        z = z + b2_ref[...]
        h = jnp.where(z >= 0, z, 0.01 * z)
        colsum = jnp.sum(h, axis=0, keepdims=True)

        @pl.when(i == 0)
        def _():
            acc_ref[...] = colsum

        @pl.when(i > 0)
        def _():
            acc_ref[...] = acc_ref[...] + colsum

        @pl.when(i == pl.num_programs(0) - 1)
        def _():
            o_ref[...] = (
                jnp.dot(acc_ref[...], w3_ref[...], preferred_element_type=jnp.float32)
                + b3_ref[...]
            )

    return pl.pallas_call(
        body,
        grid=grid,
        in_specs=[
            pl.BlockSpec((C_in, BN, _LANE), lambda i: (0, i, 0)),
            pl.BlockSpec((C_in, _LANE, D_h), lambda i: (0, 0, 0)),
            pl.BlockSpec((1, D_h), lambda i: (0, 0)),
            pl.BlockSpec((D_h, n_classes), lambda i: (0, 0)),
            pl.BlockSpec((1, n_classes), lambda i: (0, 0)),
        ],
        out_specs=pl.BlockSpec((1, n_classes), lambda i: (0, 0)),
        out_shape=jax.ShapeDtypeStruct((1, n_classes), jnp.float32),
        scratch_shapes=[pltpu.VMEM((1, D_h), jnp.float32)],
    )


def kernel(in_feat, edge_index, W1, b1, W2, b2, W3, b3):
    N, D_in = in_feat.shape
    E = edge_index.shape[1]
    D_h = W1.shape[1]
    n_classes = W3.shape[1]
    C1 = D_in // _LANE
    C2 = D_h // _LANE

    src = edge_index[0].astype(jnp.int32)
    dst = edge_index[1].astype(jnp.int32)

    e_per_s = E // _NS
    B = _edge_block(e_per_s)
    nblk = e_per_s // B

    offs1 = (jnp.arange(C1, dtype=jnp.int32) * N)[:, None]
    offs2 = (jnp.arange(C2, dtype=jnp.int32) * N)[:, None]
    src_o1 = (src[None, :] + offs1).reshape(C1 * _NS, nblk, B)
    src_o2 = (src[None, :] + offs2).reshape(C2 * _NS, nblk, B)
    dst3 = dst.reshape(_NS, nblk, B)

    xc = jnp.transpose(in_feat.reshape(N, C1, _LANE), (1, 0, 2)).reshape(C1 * N, _LANE)

    s1 = _seg_accum(C1, N, E)(xc, src_o1, dst3)
    h1 = _gin_linear(C1, C2, N, 2000)(
        s1.reshape(C1, N, _LANE),
        W1.reshape(C1, _LANE, D_h),
        b1.reshape(1, D_h),
    )
    s2 = _seg_accum(C2, N, E)(h1.reshape(C2 * N, _LANE), src_o2, dst3)
    out = _gin_final(C2, N, 2000, n_classes)(
        s2.reshape(C2, N, _LANE),
        W2.reshape(C2, _LANE, D_h),
        b2.reshape(1, D_h),
        W3,
        b3.reshape(1, n_classes),
    )
    return out


# speedup vs baseline: 1.2528x; 1.2528x over previous
import functools

import jax
import jax.numpy as jnp
from jax import lax
from jax.experimental import pallas as pl
from jax.experimental.pallas import tpu as pltpu
from jax.experimental.pallas import tpu_sc as plsc

_NC = 2
_NS = 16
_LANE = 128


def _edge_block(e_per_s: int) -> int:
    for b in (128, 120, 112, 104, 96, 88, 80, 72, 64, 56, 48, 40, 32, 24, 16, 8):
        if e_per_s % b == 0:
            return b
    return 0


@functools.lru_cache(maxsize=None)
def _seg_accum(C: int, N: int, E: int):
    assert C % _NC == 0 and E % _NS == 0
    cpc = C // _NC
    e_per_s = E // _NS
    B = _edge_block(e_per_s)
    assert B > 0
    nblk = e_per_s // B
    ph0 = (nblk // 2 + 7) // 8 * 8
    if ph0 >= nblk:
        ph0 = nblk - 8
    phases = ((0, ph0), (ph0, nblk - ph0))
    ph_max = max(ph0, nblk - ph0)

    mesh = plsc.VectorSubcoreMesh(
        core_axis_name="c", subcore_axis_name="s", num_cores=_NC, num_subcores=_NS
    )

    n_hi = ((N // _NS) + 7) // 8 * 8
    n_lo = N - (_NS - 1) * n_hi
    assert n_lo > 0 and n_lo % 8 == 0

    def _row_copy(src_ref, dst_ref, src_base, dst_base, sub):
        @pl.when(sub < _NS - 1)
        def _():
            s = pl.multiple_of(src_base + sub * n_hi, 8)
            d = pl.multiple_of(dst_base + sub * n_hi, 8)
            pltpu.sync_copy(src_ref.at[pl.ds(s, n_hi)], dst_ref.at[pl.ds(d, n_hi)])

        @pl.when(sub == _NS - 1)
        def _():
            s = pl.multiple_of(src_base + (_NS - 1) * n_hi, 8)
            d = pl.multiple_of(dst_base + (_NS - 1) * n_hi, 8)
            pltpu.sync_copy(src_ref.at[pl.ds(s, n_lo)], dst_ref.at[pl.ds(d, n_lo)])

    @functools.partial(
        pl.kernel,
        out_type=jax.ShapeDtypeStruct((C * N, _LANE), jnp.float32),
        mesh=mesh,
        scratch_types=[
            pltpu.VMEM((ph_max, B), jnp.int32),
            pltpu.VMEM((nblk, B), jnp.int32),
            pltpu.VMEM((B, _LANE), jnp.float32),
            pltpu.VMEM((B, _LANE), jnp.float32),
            pltpu.VMEM_SHARED((N, _LANE), jnp.float32),
            pltpu.SemaphoreType.DMA,
            pltpu.SemaphoreType.DMA,
        ],
    )
    def seg_kernel(x_hbm, srco_hbm, dst_hbm, out_hbm, src_v, dst_v,
                   rows0, rows1, acc_sh, gs0, gs1):
        core = lax.axis_index("c")
        sub = lax.axis_index("s")
        pltpu.sync_copy(dst_hbm.at[sub], dst_v)

        def pipe_phase(base, n, c):
            def g_start(i, buf, sem):
                pltpu.async_copy(x_hbm.at[src_v.at[i]], buf, sem)

            def g_wait(i, buf, sem):
                pltpu.make_async_copy(x_hbm.at[src_v.at[i]], buf, sem).wait()

            def scat(i, buf):
                pltpu.sync_copy(buf, acc_sh.at[dst_v.at[base + i]], add=True)

            g_start(0, rows0, gs0)
            g_start(1, rows1, gs1)
            n_pairs = (n - 2) // 2

            @pl.loop(0, n_pairs)
            def _(t):
                i = t * 2
                g_wait(i, rows0, gs0)
                scat(i, rows0)
                g_start(i + 2, rows0, gs0)
                g_wait(i + 1, rows1, gs1)
                scat(i + 1, rows1)
                g_start(i + 3, rows1, gs1)

            r = 2 * n_pairs
            g_wait(r, rows0, gs0)
            scat(r, rows0)
            if n - r == 3:
                g_start(r + 2, rows0, gs0)
            g_wait(r + 1, rows1, gs1)
            scat(r + 1, rows1)
            if n - r == 3:
                g_wait(r + 2, rows0, gs0)
                scat(r + 2, rows0)

        for j in range(cpc):
            c = core * cpc + j
            _row_copy(x_hbm, acc_sh, c * N, 0, sub)
            plsc.subcore_barrier()

            for base, n_ph in phases:
                pltpu.sync_copy(
                    srco_hbm.at[c * _NS + sub].at[pl.ds(base, n_ph)],
                    src_v.at[pl.ds(0, n_ph)],
                )
                pipe_phase(base, n_ph, c)

            plsc.subcore_barrier()
            _row_copy(acc_sh, out_hbm, 0, c * N, sub)

    return seg_kernel


@functools.lru_cache(maxsize=None)
def _gin_linear(C_in: int, C_out: int, N: int, BN: int):
    D_out = C_out * _LANE
    grid = (N // BN,)

    def body(s_ref, w_ref, b_ref, o_ref):
        acc = jnp.dot(s_ref[0], w_ref[0], preferred_element_type=jnp.float32)
        for c in range(1, C_in):
            acc += jnp.dot(s_ref[c], w_ref[c], preferred_element_type=jnp.float32)
        acc = acc + b_ref[...]
        h = jnp.where(acc >= 0, acc, 0.01 * acc)
        for j in range(C_out):
            o_ref[j] = h[:, j * _LANE:(j + 1) * _LANE]

    return pl.pallas_call(
        body,
        grid=grid,
        in_specs=[
            pl.BlockSpec((C_in, BN, _LANE), lambda i: (0, i, 0)),
            pl.BlockSpec((C_in, _LANE, D_out), lambda i: (0, 0, 0)),
            pl.BlockSpec((1, D_out), lambda i: (0, 0)),
        ],
        out_specs=pl.BlockSpec((C_out, BN, _LANE), lambda i: (0, i, 0)),
        out_shape=jax.ShapeDtypeStruct((C_out, N, _LANE), jnp.float32),
    )


@functools.lru_cache(maxsize=None)
def _gin_final(C_in: int, N: int, BN: int, n_classes: int):
    D_h = 512
    grid = (N // BN,)

    def body(s_ref, w2_ref, b2_ref, w3_ref, b3_ref, o_ref, acc_ref):
        i = pl.program_id(0)
        z = jnp.dot(s_ref[0], w2_ref[0], preferred_element_type=jnp.float32)
        for c in range(1, C_in):
            z += jnp.dot(s_ref[c], w2_ref[c], preferred_element_type=jnp.float32)
        z = z + b2_ref[...]
        h = jnp.where(z >= 0, z, 0.01 * z)
        colsum = jnp.sum(h, axis=0, keepdims=True)

        @pl.when(i == 0)
        def _():
            acc_ref[...] = colsum

        @pl.when(i > 0)
        def _():
            acc_ref[...] = acc_ref[...] + colsum

        @pl.when(i == pl.num_programs(0) - 1)
        def _():
            o_ref[...] = (
                jnp.dot(acc_ref[...], w3_ref[...], preferred_element_type=jnp.float32)
                + b3_ref[...]
            )

    return pl.pallas_call(
        body,
        grid=grid,
        in_specs=[
            pl.BlockSpec((C_in, BN, _LANE), lambda i: (0, i, 0)),
            pl.BlockSpec((C_in, _LANE, D_h), lambda i: (0, 0, 0)),
            pl.BlockSpec((1, D_h), lambda i: (0, 0)),
            pl.BlockSpec((D_h, n_classes), lambda i: (0, 0)),
            pl.BlockSpec((1, n_classes), lambda i: (0, 0)),
        ],
        out_specs=pl.BlockSpec((1, n_classes), lambda i: (0, 0)),
        out_shape=jax.ShapeDtypeStruct((1, n_classes), jnp.float32),
        scratch_shapes=[pltpu.VMEM((1, D_h), jnp.float32)],
    )


def kernel(in_feat, edge_index, W1, b1, W2, b2, W3, b3):
    N, D_in = in_feat.shape
    E = edge_index.shape[1]
    D_h = W1.shape[1]
    n_classes = W3.shape[1]
    C1 = D_in // _LANE
    C2 = D_h // _LANE

    src = edge_index[0].astype(jnp.int32)
    dst = edge_index[1].astype(jnp.int32)

    e_per_s = E // _NS
    B = _edge_block(e_per_s)
    nblk = e_per_s // B

    offs1 = (jnp.arange(C1, dtype=jnp.int32) * N)[:, None]
    offs2 = (jnp.arange(C2, dtype=jnp.int32) * N)[:, None]
    src_o1 = (src[None, :] + offs1).reshape(C1 * _NS, nblk, B)
    src_o2 = (src[None, :] + offs2).reshape(C2 * _NS, nblk, B)
    dst3 = dst.reshape(_NS, nblk, B)

    xc = jnp.transpose(in_feat.reshape(N, C1, _LANE), (1, 0, 2)).reshape(C1 * N, _LANE)

    s1 = _seg_accum(C1, N, E)(xc, src_o1, dst3)
    h1 = _gin_linear(C1, C2, N, 2000)(
        s1.reshape(C1, N, _LANE),
        W1.reshape(C1, _LANE, D_h),
        b1.reshape(1, D_h),
    )
    s2 = _seg_accum(C2, N, E)(h1.reshape(C2 * N, _LANE), src_o2, dst3)
    out = _gin_final(C2, N, 2000, n_classes)(
        s2.reshape(C2, N, _LANE),
        W2.reshape(C2, _LANE, D_h),
        b2.reshape(1, D_h),
        W3,
        b3.reshape(1, n_classes),
    )
    return out
